# Initial kernel scaffold; baseline (speedup 1.0000x reference)
import jax
import jax.numpy as jnp
from jax.experimental import pallas as pl


def kernel(x, edge_index, batch, conv1_Wrel, conv1_Wroot, conv1_b, conv2_Wrel, conv2_Wroot, conv2_b, conv3_Wrel, conv3_Wroot, conv3_b, mlp1_W, mlp1_b, mlp2_W, mlp2_b, mlp3_W, mlp3_b):
    raise NotImplementedError("write your pallas kernel here")


# speedup vs baseline: 3.0669x; 3.0669x over previous
import functools

import jax
import jax.numpy as jnp
from jax import lax
from jax.experimental import pallas as pl
from jax.experimental.pallas import tpu as pltpu
from jax.experimental.pallas import tpu_sc as plsc

N = 10000
E = 320000
D = 128
H = 64
C = 10
G = 64

NPAD = 10240
ROWS_PER_TILE = NPAD // 16
CH = 80
EPAD = 32 * CH * 128
BLK = 1024
NBLK = NPAD // BLK

_HIGH = jax.lax.Precision.HIGHEST


def _dot(a, b):
    return jnp.dot(a, b, precision=_HIGH, preferred_element_type=jnp.float32)


@functools.lru_cache(maxsize=None)
def _make_segsum(F):
    mesh = plsc.VectorSubcoreMesh(core_axis_name="c", subcore_axis_name="s")

    @functools.partial(
        pl.kernel,
        out_type=jax.ShapeDtypeStruct((2, NPAD, F), jnp.float32),
        mesh=mesh,
        scratch_types=[
            pltpu.VMEM((CH, 128), jnp.int32),
            pltpu.VMEM((CH, 128), jnp.int32),
            pltpu.VMEM((128, F), jnp.float32),
            pltpu.VMEM_SHARED((NPAD, F), jnp.float32),
            pltpu.SemaphoreType.DMA,
        ],
    )
    def segsum(table_hbm, srcm_hbm, dstm_hbm, zeros_hbm, out_hbm,
               src_v, dst_v, rows_v, acc_sh, gsem):
        c = lax.axis_index("c")
        s = lax.axis_index("s")
        tile = c * 16 + s

        pltpu.sync_copy(zeros_hbm, acc_sh.at[pl.ds(s * ROWS_PER_TILE, ROWS_PER_TILE)])

        pltpu.sync_copy(srcm_hbm.at[pl.ds(tile * CH, CH)], src_v)
        pltpu.sync_copy(dstm_hbm.at[pl.ds(tile * CH, CH)], dst_v)
        plsc.subcore_barrier()

        def body(j, carry):
            pltpu.async_copy(table_hbm.at[src_v.at[j]], rows_v, gsem).wait()
            pltpu.sync_copy(rows_v, acc_sh.at[dst_v.at[j]], add=True)
            return carry

        lax.fori_loop(0, CH, body, 0)
        plsc.subcore_barrier()

        rows = pl.ds(s * ROWS_PER_TILE, ROWS_PER_TILE)
        pltpu.sync_copy(acc_sh.at[rows], out_hbm.at[c, rows])

    return segsum


def _pre_body(x_ref, w_ref, o_ref):
    o_ref[...] = _dot(x_ref[...], w_ref[...])


def _pre(x_pad, w):
    return pl.pallas_call(
        _pre_body,
        grid=(NBLK,),
        in_specs=[
            pl.BlockSpec((BLK, x_pad.shape[1]), lambda i: (i, 0)),
            pl.BlockSpec(w.shape, lambda i: (0, 0)),
        ],
        out_specs=pl.BlockSpec((BLK, w.shape[1]), lambda i: (i, 0)),
        out_shape=jax.ShapeDtypeStruct((NPAD, w.shape[1]), jnp.float32),
    )(x_pad, w)


def _mid1_body(a0_ref, a1_ref, x_ref, wr_ref, b_ref, o_ref):
    v = a0_ref[...] + a1_ref[...] + _dot(x_ref[...], wr_ref[...]) + b_ref[...]
    o_ref[...] = jnp.where(v > 0, v, jnp.exp(jnp.minimum(v, 0.0)) - 1.0)


def _mid1(a0, a1, x_pad, wroot, b):
    F = wroot.shape[1]
    return pl.pallas_call(
        _mid1_body,
        grid=(NBLK,),
        in_specs=[
            pl.BlockSpec((BLK, F), lambda i: (i, 0)),
            pl.BlockSpec((BLK, F), lambda i: (i, 0)),
            pl.BlockSpec((BLK, x_pad.shape[1]), lambda i: (i, 0)),
            pl.BlockSpec(wroot.shape, lambda i: (0, 0)),
            pl.BlockSpec((1, F), lambda i: (0, 0)),
        ],
        out_specs=pl.BlockSpec((BLK, F), lambda i: (i, 0)),
        out_shape=jax.ShapeDtypeStruct((NPAD, F), jnp.float32),
    )(a0, a1, x_pad, wroot, b.reshape(1, F))


def _mid_body(a0_ref, a1_ref, h_ref, wrel_ref, wroot_ref, b_ref, o_ref):
    agg = a0_ref[...] + a1_ref[...]
    v = _dot(agg, wrel_ref[...]) + _dot(h_ref[...], wroot_ref[...]) + b_ref[...]
    o_ref[...] = jnp.where(v > 0, v, jnp.exp(jnp.minimum(v, 0.0)) - 1.0)


def _mid(a0, a1, h, wrel, wroot, b):
    Fi = h.shape[1]
    Fo = wrel.shape[1]
    return pl.pallas_call(
        _mid_body,
        grid=(NBLK,),
        in_specs=[
            pl.BlockSpec((BLK, Fi), lambda i: (i, 0)),
            pl.BlockSpec((BLK, Fi), lambda i: (i, 0)),
            pl.BlockSpec((BLK, Fi), lambda i: (i, 0)),
            pl.BlockSpec(wrel.shape, lambda i: (0, 0)),
            pl.BlockSpec(wroot.shape, lambda i: (0, 0)),
            pl.BlockSpec((1, Fo), lambda i: (0, 0)),
        ],
        out_specs=pl.BlockSpec((BLK, Fo), lambda i: (i, 0)),
        out_shape=jax.ShapeDtypeStruct((NPAD, Fo), jnp.float32),
    )(a0, a1, h, wrel, wroot, b.reshape(1, Fo))


def _pool_body(a0_ref, a1_ref, h_ref, wrel_ref, wroot_ref, b_ref, bat_ref,
               sums_ref, cnts_ref):
    i = pl.program_id(0)
    agg = a0_ref[...] + a1_ref[...]
    v = _dot(agg, wrel_ref[...]) + _dot(h_ref[...], wroot_ref[...]) + b_ref[...]
    h3 = jnp.where(v > 0, v, jnp.exp(jnp.minimum(v, 0.0)) - 1.0)
    bat = bat_ref[pl.ds(i * BLK, BLK)]
    onehot = (bat.reshape(BLK, 1) ==
              lax.broadcasted_iota(jnp.int32, (1, G), 1)).astype(jnp.float32)

    @pl.when(i == 0)
    def _():
        sums_ref[...] = jnp.zeros_like(sums_ref)
        cnts_ref[...] = jnp.zeros_like(cnts_ref)

    sums_ref[...] += _dot(onehot.T, h3)
    cnts_ref[...] += jnp.sum(onehot, axis=0).reshape(G, 1) + jnp.zeros(
        (G, 128), jnp.float32)


def _pool(a0, a1, h, wrel, wroot, b, batch_pad):
    Fi = h.shape[1]
    Fo = wrel.shape[1]
    return pl.pallas_call(
        _pool_body,
        grid=(NBLK,),
        in_specs=[
            pl.BlockSpec((BLK, Fi), lambda i: (i, 0)),
            pl.BlockSpec((BLK, Fi), lambda i: (i, 0)),
            pl.BlockSpec((BLK, Fi), lambda i: (i, 0)),
            pl.BlockSpec(wrel.shape, lambda i: (0, 0)),
            pl.BlockSpec(wroot.shape, lambda i: (0, 0)),
            pl.BlockSpec((1, Fo), lambda i: (0, 0)),
            pl.BlockSpec((NPAD,), lambda i: (0,)),
        ],
        out_specs=[
            pl.BlockSpec((G, Fo), lambda i: (0, 0)),
            pl.BlockSpec((G, 128), lambda i: (0, 0)),
        ],
        out_shape=[
            jax.ShapeDtypeStruct((G, Fo), jnp.float32),
            jax.ShapeDtypeStruct((G, 128), jnp.float32),
        ],
    )(a0, a1, h, wrel, wroot, b.reshape(1, Fo), batch_pad)


def _head_body(sums_ref, cnts_ref, w1_ref, b1_ref, w2_ref, b2_ref,
               w3_ref, b3_ref, o_ref):
    cnt = jnp.maximum(cnts_ref[:, 0:1], 1.0)
    m = sums_ref[...] / cnt
    z = _dot(m, w1_ref[...]) + b1_ref[...]
    z = jnp.where(z > 0, z, jnp.exp(jnp.minimum(z, 0.0)) - 1.0)
    z = _dot(z, w2_ref[...]) + b2_ref[...]
    z = jnp.where(z > 0, z, jnp.exp(jnp.minimum(z, 0.0)) - 1.0)
    z = _dot(z, w3_ref[...]) + b3_ref[...]
    zmax = jnp.max(z, axis=1, keepdims=True)
    lse = zmax + jnp.log(jnp.sum(jnp.exp(z - zmax), axis=1, keepdims=True))
    o_ref[...] = z - lse


def _head(sums, cnts, w1, b1, w2, b2, w3, b3):
    return pl.pallas_call(
        _head_body,
        out_shape=jax.ShapeDtypeStruct((G, C), jnp.float32),
    )(sums, cnts, w1, b1.reshape(1, -1), w2, b2.reshape(1, -1),
      w3, b3.reshape(1, -1))


def kernel(x, edge_index, batch,
           conv1_Wrel, conv1_Wroot, conv1_b,
           conv2_Wrel, conv2_Wroot, conv2_b,
           conv3_Wrel, conv3_Wroot, conv3_b,
           mlp1_W, mlp1_b, mlp2_W, mlp2_b, mlp3_W, mlp3_b):
    x_pad = jnp.pad(x, ((0, NPAD - N), (0, 0)))
    batch_pad = jnp.pad(batch, (0, NPAD - N), constant_values=G)

    src = jnp.pad(edge_index[0], (0, EPAD - E)).reshape(32 * CH, 128)
    dst = jnp.pad(edge_index[1], (0, EPAD - E),
                  constant_values=N).reshape(32 * CH, 128)
    zeros128 = jnp.zeros((ROWS_PER_TILE, 128), jnp.float32)

    w1rel = jnp.pad(conv1_Wrel, ((0, 0), (0, 64)))
    w1root = jnp.pad(conv1_Wroot, ((0, 0), (0, 64)))
    b1 = jnp.pad(conv1_b, (0, 64))
    w2rel = jnp.pad(conv2_Wrel, ((0, 64), (0, 0)))
    w2root = jnp.pad(conv2_Wroot, ((0, 64), (0, 0)))

    y1 = _pre(x_pad, w1rel)
    p1 = _make_segsum(128)(y1, src, dst, zeros128)
    h1 = _mid1(p1[0], p1[1], x_pad, w1root, b1)

    p2 = _make_segsum(128)(h1, src, dst, zeros128)
    h2 = _mid(p2[0], p2[1], h1, w2rel, w2root, conv2_b)

    p3 = _make_segsum(128)(h2, src, dst, zeros128)
    sums, cnts = _pool(p3[0], p3[1], h2, conv3_Wrel, conv3_Wroot, conv3_b,
                       batch_pad)

    return _head(sums, cnts, mlp1_W, mlp1_b, mlp2_W, mlp2_b, mlp3_W, mlp3_b)


# speedup vs baseline: 3.2806x; 1.0697x over previous
import functools

import jax
import jax.numpy as jnp
from jax import lax
from jax.experimental import pallas as pl
from jax.experimental.pallas import tpu as pltpu
from jax.experimental.pallas import tpu_sc as plsc

N = 10000
E = 320000
D = 128
H = 64
C = 10
G = 64

NPAD = 10240
ROWS_PER_TILE = NPAD // 16
CH = 80
EPAD = 32 * CH * 128
GS = 16
NG = CH // GS
BLK = 1024
NBLK = NPAD // BLK

_HIGH = jax.lax.Precision.HIGHEST


def _dot(a, b):
    return jnp.dot(a, b, precision=_HIGH, preferred_element_type=jnp.float32)


@functools.lru_cache(maxsize=None)
def _make_segsum(F):
    mesh = plsc.VectorSubcoreMesh(core_axis_name="c", subcore_axis_name="s")

    @functools.partial(
        pl.kernel,
        out_type=jax.ShapeDtypeStruct((2, NPAD, F), jnp.float32),
        mesh=mesh,
        scratch_types=[
            pltpu.VMEM((GS, 128), jnp.int32),
            pltpu.VMEM((GS, 128), jnp.int32),
            pltpu.VMEM((2 * 128, F), jnp.float32),
            pltpu.VMEM_SHARED((NPAD, F), jnp.float32),
            pltpu.SemaphoreType.DMA,
        ],
    )
    def segsum(table_hbm, srcm_hbm, dstm_hbm, zeros_hbm, out_hbm,
               src_v, dst_v, rows_v, acc_sh, gsem):
        c = lax.axis_index("c")
        s = lax.axis_index("s")
        tile = c * 16 + s

        pltpu.sync_copy(zeros_hbm, acc_sh.at[pl.ds(s * ROWS_PER_TILE, ROWS_PER_TILE)])
        plsc.subcore_barrier()

        def group_body(g, carry):
            base = tile * CH + g * GS
            pltpu.sync_copy(srcm_hbm.at[pl.ds(base, GS)], src_v)
            pltpu.sync_copy(dstm_hbm.at[pl.ds(base, GS)], dst_v)
            pltpu.async_copy(table_hbm.at[src_v.at[0]],
                             rows_v.at[pl.ds(0, 128)], gsem)

            def chunk_body(k, carry2):
                r0 = (k % 2) * 128
                r1 = 128 - r0
                pltpu.make_async_copy(table_hbm.at[src_v.at[k]],
                                      rows_v.at[pl.ds(r0, 128)], gsem).wait()

                @pl.when(k < GS - 1)
                def _():
                    pltpu.async_copy(table_hbm.at[src_v.at[k + 1]],
                                     rows_v.at[pl.ds(r1, 128)], gsem)

                pltpu.sync_copy(rows_v.at[pl.ds(r0, 128)],
                                acc_sh.at[dst_v.at[k]], add=True)
                return carry2

            lax.fori_loop(0, GS, chunk_body, 0)
            return carry

        lax.fori_loop(0, NG, group_body, 0)
        plsc.subcore_barrier()

        rows = pl.ds(s * ROWS_PER_TILE, ROWS_PER_TILE)
        pltpu.sync_copy(acc_sh.at[rows], out_hbm.at[c, rows])

    return segsum


def _pre_body(x_ref, w_ref, o_ref):
    o_ref[...] = _dot(x_ref[...], w_ref[...])


def _pre(x_pad, w):
    return pl.pallas_call(
        _pre_body,
        grid=(NBLK,),
        in_specs=[
            pl.BlockSpec((BLK, x_pad.shape[1]), lambda i: (i, 0)),
            pl.BlockSpec(w.shape, lambda i: (0, 0)),
        ],
        out_specs=pl.BlockSpec((BLK, w.shape[1]), lambda i: (i, 0)),
        out_shape=jax.ShapeDtypeStruct((NPAD, w.shape[1]), jnp.float32),
    )(x_pad, w)


def _mid1_body(a0_ref, a1_ref, x_ref, wr_ref, b_ref, o_ref):
    v = a0_ref[...] + a1_ref[...] + _dot(x_ref[...], wr_ref[...]) + b_ref[...]
    o_ref[...] = jnp.where(v > 0, v, jnp.exp(jnp.minimum(v, 0.0)) - 1.0)


def _mid1(a0, a1, x_pad, wroot, b):
    F = wroot.shape[1]
    return pl.pallas_call(
        _mid1_body,
        grid=(NBLK,),
        in_specs=[
            pl.BlockSpec((BLK, F), lambda i: (i, 0)),
            pl.BlockSpec((BLK, F), lambda i: (i, 0)),
            pl.BlockSpec((BLK, x_pad.shape[1]), lambda i: (i, 0)),
            pl.BlockSpec(wroot.shape, lambda i: (0, 0)),
            pl.BlockSpec((1, F), lambda i: (0, 0)),
        ],
        out_specs=pl.BlockSpec((BLK, F), lambda i: (i, 0)),
        out_shape=jax.ShapeDtypeStruct((NPAD, F), jnp.float32),
    )(a0, a1, x_pad, wroot, b.reshape(1, F))


def _mid_body(a0_ref, a1_ref, h_ref, wrel_ref, wroot_ref, b_ref, o_ref):
    agg = a0_ref[...] + a1_ref[...]
    v = _dot(agg, wrel_ref[...]) + _dot(h_ref[...], wroot_ref[...]) + b_ref[...]
    o_ref[...] = jnp.where(v > 0, v, jnp.exp(jnp.minimum(v, 0.0)) - 1.0)


def _mid(a0, a1, h, wrel, wroot, b):
    Fi = h.shape[1]
    Fo = wrel.shape[1]
    return pl.pallas_call(
        _mid_body,
        grid=(NBLK,),
        in_specs=[
            pl.BlockSpec((BLK, Fi), lambda i: (i, 0)),
            pl.BlockSpec((BLK, Fi), lambda i: (i, 0)),
            pl.BlockSpec((BLK, Fi), lambda i: (i, 0)),
            pl.BlockSpec(wrel.shape, lambda i: (0, 0)),
            pl.BlockSpec(wroot.shape, lambda i: (0, 0)),
            pl.BlockSpec((1, Fo), lambda i: (0, 0)),
        ],
        out_specs=pl.BlockSpec((BLK, Fo), lambda i: (i, 0)),
        out_shape=jax.ShapeDtypeStruct((NPAD, Fo), jnp.float32),
    )(a0, a1, h, wrel, wroot, b.reshape(1, Fo))


def _pool_body(a0_ref, a1_ref, h_ref, wrel_ref, wroot_ref, b_ref, bat_ref,
               sums_ref, cnts_ref):
    i = pl.program_id(0)
    agg = a0_ref[...] + a1_ref[...]
    v = _dot(agg, wrel_ref[...]) + _dot(h_ref[...], wroot_ref[...]) + b_ref[...]
    h3 = jnp.where(v > 0, v, jnp.exp(jnp.minimum(v, 0.0)) - 1.0)
    bat = bat_ref[pl.ds(i * BLK, BLK)]
    onehot = (bat.reshape(BLK, 1) ==
              lax.broadcasted_iota(jnp.int32, (1, G), 1)).astype(jnp.float32)

    @pl.when(i == 0)
    def _():
        sums_ref[...] = jnp.zeros_like(sums_ref)
        cnts_ref[...] = jnp.zeros_like(cnts_ref)

    sums_ref[...] += _dot(onehot.T, h3)
    cnts_ref[...] += jnp.sum(onehot, axis=0).reshape(G, 1) + jnp.zeros(
        (G, 128), jnp.float32)


def _pool(a0, a1, h, wrel, wroot, b, batch_pad):
    Fi = h.shape[1]
    Fo = wrel.shape[1]
    return pl.pallas_call(
        _pool_body,
        grid=(NBLK,),
        in_specs=[
            pl.BlockSpec((BLK, Fi), lambda i: (i, 0)),
            pl.BlockSpec((BLK, Fi), lambda i: (i, 0)),
            pl.BlockSpec((BLK, Fi), lambda i: (i, 0)),
            pl.BlockSpec(wrel.shape, lambda i: (0, 0)),
            pl.BlockSpec(wroot.shape, lambda i: (0, 0)),
            pl.BlockSpec((1, Fo), lambda i: (0, 0)),
            pl.BlockSpec((NPAD,), lambda i: (0,)),
        ],
        out_specs=[
            pl.BlockSpec((G, Fo), lambda i: (0, 0)),
            pl.BlockSpec((G, 128), lambda i: (0, 0)),
        ],
        out_shape=[
            jax.ShapeDtypeStruct((G, Fo), jnp.float32),
            jax.ShapeDtypeStruct((G, 128), jnp.float32),
        ],
    )(a0, a1, h, wrel, wroot, b.reshape(1, Fo), batch_pad)


def _head_body(sums_ref, cnts_ref, w1_ref, b1_ref, w2_ref, b2_ref,
               w3_ref, b3_ref, o_ref):
    cnt = jnp.maximum(cnts_ref[:, 0:1], 1.0)
    m = sums_ref[...] / cnt
    z = _dot(m, w1_ref[...]) + b1_ref[...]
    z = jnp.where(z > 0, z, jnp.exp(jnp.minimum(z, 0.0)) - 1.0)
    z = _dot(z, w2_ref[...]) + b2_ref[...]
    z = jnp.where(z > 0, z, jnp.exp(jnp.minimum(z, 0.0)) - 1.0)
    z = _dot(z, w3_ref[...]) + b3_ref[...]
    zmax = jnp.max(z, axis=1, keepdims=True)
    lse = zmax + jnp.log(jnp.sum(jnp.exp(z - zmax), axis=1, keepdims=True))
    o_ref[...] = z - lse


def _head(sums, cnts, w1, b1, w2, b2, w3, b3):
    return pl.pallas_call(
        _head_body,
        out_shape=jax.ShapeDtypeStruct((G, C), jnp.float32),
    )(sums, cnts, w1, b1.reshape(1, -1), w2, b2.reshape(1, -1),
      w3, b3.reshape(1, -1))


def kernel(x, edge_index, batch,
           conv1_Wrel, conv1_Wroot, conv1_b,
           conv2_Wrel, conv2_Wroot, conv2_b,
           conv3_Wrel, conv3_Wroot, conv3_b,
           mlp1_W, mlp1_b, mlp2_W, mlp2_b, mlp3_W, mlp3_b):
    x_pad = jnp.pad(x, ((0, NPAD - N), (0, 0)))
    batch_pad = jnp.pad(batch, (0, NPAD - N), constant_values=G)

    src = jnp.pad(edge_index[0], (0, EPAD - E)).reshape(32 * CH, 128)
    dst = jnp.pad(edge_index[1], (0, EPAD - E),
                  constant_values=N).reshape(32 * CH, 128)
    zeros128 = jnp.zeros((ROWS_PER_TILE, 128), jnp.float32)

    w1rel = jnp.pad(conv1_Wrel, ((0, 0), (0, 64)))
    w1root = jnp.pad(conv1_Wroot, ((0, 0), (0, 64)))
    b1 = jnp.pad(conv1_b, (0, 64))
    w2rel = jnp.pad(conv2_Wrel, ((0, 64), (0, 0)))
    w2root = jnp.pad(conv2_Wroot, ((0, 64), (0, 0)))

    y1 = _pre(x_pad, w1rel)
    p1 = _make_segsum(128)(y1, src, dst, zeros128)
    h1 = _mid1(p1[0], p1[1], x_pad, w1root, b1)

    p2 = _make_segsum(128)(h1, src, dst, zeros128)
    h2 = _mid(p2[0], p2[1], h1, w2rel, w2root, conv2_b)

    p3 = _make_segsum(128)(h2, src, dst, zeros128)
    sums, cnts = _pool(p3[0], p3[1], h2, conv3_Wrel, conv3_Wroot, conv3_b,
                       batch_pad)

    return _head(sums, cnts, mlp1_W, mlp1_b, mlp2_W, mlp2_b, mlp3_W, mlp3_b)


# speedup vs baseline: 4.4824x; 1.3663x over previous
import functools

import jax
import jax.numpy as jnp
from jax import lax
from jax.experimental import pallas as pl
from jax.experimental.pallas import tpu as pltpu
from jax.experimental.pallas import tpu_sc as plsc

N = 10000
E = 320000
D = 128
H = 64
C = 10
G = 64

NPAD = 10240
ROWS_PER_TILE = NPAD // 16
CH = 80
EPAD = 32 * CH * 128
GS = 16
NG = CH // GS
DEPTH = 4
BLK = 1024
NBLK = NPAD // BLK

_HIGH = jax.lax.Precision.HIGHEST


def _dot(a, b):
    return jnp.dot(a, b, precision=_HIGH, preferred_element_type=jnp.float32)


def _elu(v):
    return jnp.where(v > 0, v, jnp.exp(jnp.minimum(v, 0.0)) - 1.0)


@functools.lru_cache(maxsize=None)
def _make_segsum():
    F = 64
    mesh = plsc.VectorSubcoreMesh(core_axis_name="c", subcore_axis_name="s")

    @functools.partial(
        pl.kernel,
        out_type=jax.ShapeDtypeStruct((2, NPAD, F), jnp.float32),
        mesh=mesh,
        scratch_types=[
            pltpu.VMEM((GS, 128), jnp.int32),
            pltpu.VMEM((GS, 128), jnp.int32),
            pltpu.VMEM((DEPTH * 128, F), jnp.float32),
            pltpu.VMEM_SHARED((NPAD, F), jnp.float32),
            pltpu.SemaphoreType.DMA,
        ],
        compiler_params=pltpu.CompilerParams(use_tc_tiling_on_sc=False),
    )
    def segsum(table_hbm, srcm_hbm, dstm_hbm, zeros_hbm, out_hbm,
               src_v, dst_v, rows_v, acc_sh, gsem):
        c = lax.axis_index("c")
        s = lax.axis_index("s")
        tile = c * 16 + s
        rows = pl.ds(s * ROWS_PER_TILE, ROWS_PER_TILE)

        pltpu.sync_copy(zeros_hbm, acc_sh.at[rows])
        plsc.subcore_barrier()

        def group_body(g, carry):
            base = tile * CH + g * GS
            pltpu.sync_copy(srcm_hbm.at[pl.ds(base, GS)], src_v)
            pltpu.sync_copy(dstm_hbm.at[pl.ds(base, GS)], dst_v)
            for k in range(DEPTH - 1):
                pltpu.async_copy(table_hbm.at[src_v.at[k]],
                                 rows_v.at[pl.ds(k * 128, 128)], gsem)

            def chunk_body(k, carry2):
                r0 = (k % DEPTH) * 128
                pltpu.make_async_copy(table_hbm.at[src_v.at[k]],
                                      rows_v.at[pl.ds(r0, 128)], gsem).wait()

                @pl.when(k < GS - (DEPTH - 1))
                def _():
                    kn = k + DEPTH - 1
                    rn = (kn % DEPTH) * 128
                    pltpu.async_copy(table_hbm.at[src_v.at[kn]],
                                     rows_v.at[pl.ds(rn, 128)], gsem)

                pltpu.sync_copy(rows_v.at[pl.ds(r0, 128)],
                                acc_sh.at[dst_v.at[k]], add=True)
                return carry2

            lax.fori_loop(0, GS, chunk_body, 0)
            return carry

        lax.fori_loop(0, NG, group_body, 0)
        plsc.subcore_barrier()

        pltpu.sync_copy(acc_sh.at[rows], out_hbm.at[c, rows])

    return segsum


def _pre_body(x_ref, w_ref, o_ref):
    o_ref[...] = _dot(x_ref[...], w_ref[...])


def _pre(x_pad, w):
    return pl.pallas_call(
        _pre_body,
        grid=(NBLK,),
        in_specs=[
            pl.BlockSpec((BLK, x_pad.shape[1]), lambda i: (i, 0)),
            pl.BlockSpec(w.shape, lambda i: (0, 0)),
        ],
        out_specs=pl.BlockSpec((BLK, w.shape[1]), lambda i: (i, 0)),
        out_shape=jax.ShapeDtypeStruct((NPAD, w.shape[1]), jnp.float32),
    )(x_pad, w)


def _mid1_body(a0_ref, a1_ref, x_ref, wr_ref, b_ref, o_ref):
    v = a0_ref[...] + a1_ref[...] + _dot(x_ref[...], wr_ref[...]) + b_ref[...]
    o_ref[...] = _elu(v)


def _mid1(a0, a1, x_pad, wroot, b):
    F = wroot.shape[1]
    return pl.pallas_call(
        _mid1_body,
        grid=(NBLK,),
        in_specs=[
            pl.BlockSpec((BLK, F), lambda i: (i, 0)),
            pl.BlockSpec((BLK, F), lambda i: (i, 0)),
            pl.BlockSpec((BLK, x_pad.shape[1]), lambda i: (i, 0)),
            pl.BlockSpec(wroot.shape, lambda i: (0, 0)),
            pl.BlockSpec((1, F), lambda i: (0, 0)),
        ],
        out_specs=pl.BlockSpec((BLK, F), lambda i: (i, 0)),
        out_shape=jax.ShapeDtypeStruct((NPAD, F), jnp.float32),
    )(a0, a1, x_pad, wroot, b.reshape(1, F))


def _mid2_body(a0_ref, a1_ref, h_ref, wrel_ref, wroot_ref, b_ref,
               lo_ref, hi_ref):
    agg = a0_ref[...] + a1_ref[...]
    v = _dot(agg, wrel_ref[...]) + _dot(h_ref[...], wroot_ref[...]) + b_ref[...]
    h2 = _elu(v)
    lo_ref[...] = h2[:, :64]
    hi_ref[...] = h2[:, 64:]


def _mid2(a0, a1, h, wrel, wroot, b):
    return pl.pallas_call(
        _mid2_body,
        grid=(NBLK,),
        in_specs=[
            pl.BlockSpec((BLK, 64), lambda i: (i, 0)),
            pl.BlockSpec((BLK, 64), lambda i: (i, 0)),
            pl.BlockSpec((BLK, 64), lambda i: (i, 0)),
            pl.BlockSpec((64, 128), lambda i: (0, 0)),
            pl.BlockSpec((64, 128), lambda i: (0, 0)),
            pl.BlockSpec((1, 128), lambda i: (0, 0)),
        ],
        out_specs=[
            pl.BlockSpec((BLK, 64), lambda i: (i, 0)),
            pl.BlockSpec((BLK, 64), lambda i: (i, 0)),
        ],
        out_shape=[
            jax.ShapeDtypeStruct((NPAD, 64), jnp.float32),
            jax.ShapeDtypeStruct((NPAD, 64), jnp.float32),
        ],
    )(a0, a1, h, wrel, wroot, b.reshape(1, 128))


def _pool_body(a0lo_ref, a1lo_ref, a0hi_ref, a1hi_ref, hlo_ref, hhi_ref,
               wrlo_ref, wrhi_ref, wolo_ref, wohi_ref, b_ref, bat_ref,
               sums_ref, cnts_ref):
    i = pl.program_id(0)
    v = (_dot(a0lo_ref[...] + a1lo_ref[...], wrlo_ref[...])
         + _dot(a0hi_ref[...] + a1hi_ref[...], wrhi_ref[...])
         + _dot(hlo_ref[...], wolo_ref[...])
         + _dot(hhi_ref[...], wohi_ref[...])
         + b_ref[...])
    h3 = _elu(v)
    bat = bat_ref[pl.ds(i * BLK, BLK)]
    onehot = (bat.reshape(BLK, 1) ==
              lax.broadcasted_iota(jnp.int32, (1, G), 1)).astype(jnp.float32)

    @pl.when(i == 0)
    def _():
        sums_ref[...] = jnp.zeros_like(sums_ref)
        cnts_ref[...] = jnp.zeros_like(cnts_ref)

    sums_ref[...] += _dot(onehot.T, h3)
    cnts_ref[...] += jnp.sum(onehot, axis=0).reshape(G, 1) + jnp.zeros(
        (G, 128), jnp.float32)


def _pool(a0lo, a1lo, a0hi, a1hi, hlo, hhi, wrel, wroot, b, batch_pad):
    half = pl.BlockSpec((BLK, 64), lambda i: (i, 0))
    wspec = pl.BlockSpec((64, 128), lambda i: (0, 0))
    return pl.pallas_call(
        _pool_body,
        grid=(NBLK,),
        in_specs=[half, half, half, half, half, half,
                  wspec, wspec, wspec, wspec,
                  pl.BlockSpec((1, 128), lambda i: (0, 0)),
                  pl.BlockSpec((NPAD,), lambda i: (0,))],
        out_specs=[
            pl.BlockSpec((G, 128), lambda i: (0, 0)),
            pl.BlockSpec((G, 128), lambda i: (0, 0)),
        ],
        out_shape=[
            jax.ShapeDtypeStruct((G, 128), jnp.float32),
            jax.ShapeDtypeStruct((G, 128), jnp.float32),
        ],
    )(a0lo, a1lo, a0hi, a1hi, hlo, hhi,
      wrel[:64], wrel[64:], wroot[:64], wroot[64:],
      b.reshape(1, 128), batch_pad)


def _head_body(sums_ref, cnts_ref, w1_ref, b1_ref, w2_ref, b2_ref,
               w3_ref, b3_ref, o_ref):
    cnt = jnp.maximum(cnts_ref[:, 0:1], 1.0)
    m = sums_ref[...] / cnt
    z = _elu(_dot(m, w1_ref[...]) + b1_ref[...])
    z = _elu(_dot(z, w2_ref[...]) + b2_ref[...])
    z = _dot(z, w3_ref[...]) + b3_ref[...]
    zmax = jnp.max(z, axis=1, keepdims=True)
    lse = zmax + jnp.log(jnp.sum(jnp.exp(z - zmax), axis=1, keepdims=True))
    o_ref[...] = z - lse


def _head(sums, cnts, w1, b1, w2, b2, w3, b3):
    return pl.pallas_call(
        _head_body,
        out_shape=jax.ShapeDtypeStruct((G, C), jnp.float32),
    )(sums, cnts, w1, b1.reshape(1, -1), w2, b2.reshape(1, -1),
      w3, b3.reshape(1, -1))


def kernel(x, edge_index, batch,
           conv1_Wrel, conv1_Wroot, conv1_b,
           conv2_Wrel, conv2_Wroot, conv2_b,
           conv3_Wrel, conv3_Wroot, conv3_b,
           mlp1_W, mlp1_b, mlp2_W, mlp2_b, mlp3_W, mlp3_b):
    x_pad = jnp.pad(x, ((0, NPAD - N), (0, 0)))
    batch_pad = jnp.pad(batch, (0, NPAD - N), constant_values=G)

    src = jnp.pad(edge_index[0], (0, EPAD - E)).reshape(32 * CH, 128)
    dst = jnp.pad(edge_index[1], (0, EPAD - E),
                  constant_values=N).reshape(32 * CH, 128)
    zeros64 = jnp.zeros((ROWS_PER_TILE, 64), jnp.float32)

    seg = _make_segsum()

    y1 = _pre(x_pad, conv1_Wrel)
    p1 = seg(y1, src, dst, zeros64)
    h1 = _mid1(p1[0], p1[1], x_pad, conv1_Wroot, conv1_b)

    p2 = seg(h1, src, dst, zeros64)
    h2lo, h2hi = _mid2(p2[0], p2[1], h1, conv2_Wrel, conv2_Wroot, conv2_b)

    p3lo = seg(h2lo, src, dst, zeros64)
    p3hi = seg(h2hi, src, dst, zeros64)
    sums, cnts = _pool(p3lo[0], p3lo[1], p3hi[0], p3hi[1], h2lo, h2hi,
                       conv3_Wrel, conv3_Wroot, conv3_b, batch_pad)

    return _head(sums, cnts, mlp1_W, mlp1_b, mlp2_W, mlp2_b, mlp3_W, mlp3_b)


# speedup vs baseline: 4.4860x; 1.0008x over previous
import functools

import jax
import jax.numpy as jnp
from jax import lax
from jax.experimental import pallas as pl
from jax.experimental.pallas import tpu as pltpu
from jax.experimental.pallas import tpu_sc as plsc

N = 10000
E = 320000
D = 128
H = 64
C = 10
G = 64

NPAD = 10240
ROWS_PER_TILE = NPAD // 16
CH = 80
EPAD = 32 * CH * 128
GS = 16
NG = CH // GS
DEPTH = 4
BLK = 1024
NBLK = NPAD // BLK

_HIGH = jax.lax.Precision.HIGHEST


def _dot(a, b):
    return jnp.dot(a, b, precision=_HIGH, preferred_element_type=jnp.float32)


def _elu(v):
    return jnp.where(v > 0, v, jnp.exp(jnp.minimum(v, 0.0)) - 1.0)


@functools.lru_cache(maxsize=None)
def _make_segsum():
    F = 64
    mesh = plsc.VectorSubcoreMesh(core_axis_name="c", subcore_axis_name="s")

    @functools.partial(
        pl.kernel,
        out_type=jax.ShapeDtypeStruct((2, NPAD, F), jnp.float32),
        mesh=mesh,
        scratch_types=[
            pltpu.VMEM((GS, 128), jnp.int32),
            pltpu.VMEM((GS, 128), jnp.int32),
            pltpu.VMEM((DEPTH * 128, F), jnp.float32),
            pltpu.VMEM_SHARED((NPAD, F), jnp.float32),
            pltpu.SemaphoreType.DMA,
            pltpu.SemaphoreType.DMA,
        ],
        compiler_params=pltpu.CompilerParams(use_tc_tiling_on_sc=False),
    )
    def segsum(table_hbm, srcm_hbm, dstm_hbm, zeros_hbm, out_hbm,
               src_v, dst_v, rows_v, acc_sh, gsem, ssem):
        c = lax.axis_index("c")
        s = lax.axis_index("s")
        tile = c * 16 + s
        rows = pl.ds(s * ROWS_PER_TILE, ROWS_PER_TILE)

        pltpu.sync_copy(zeros_hbm, acc_sh.at[rows])
        plsc.subcore_barrier()

        def group_body(g, carry):
            base = tile * CH + g * GS
            pltpu.sync_copy(srcm_hbm.at[pl.ds(base, GS)], src_v)
            pltpu.sync_copy(dstm_hbm.at[pl.ds(base, GS)], dst_v)
            for k in range(DEPTH - 1):
                pltpu.async_copy(table_hbm.at[src_v.at[k]],
                                 rows_v.at[pl.ds(k * 128, 128)], gsem)

            def chunk_body(k, carry2):
                r0 = (k % DEPTH) * 128
                pltpu.make_async_copy(table_hbm.at[src_v.at[k]],
                                      rows_v.at[pl.ds(r0, 128)], gsem).wait()
                pltpu.async_copy(rows_v.at[pl.ds(r0, 128)],
                                 acc_sh.at[dst_v.at[k]], ssem, add=True)
                kn = k + DEPTH - 1

                @pl.when((kn < GS) & (k >= 1))
                def _():
                    rp = ((k - 1) % DEPTH) * 128
                    pltpu.make_async_copy(rows_v.at[pl.ds(rp, 128)],
                                          acc_sh.at[dst_v.at[k - 1]],
                                          ssem).wait()

                @pl.when(kn < GS)
                def _():
                    rn = (kn % DEPTH) * 128
                    pltpu.async_copy(table_hbm.at[src_v.at[kn]],
                                     rows_v.at[pl.ds(rn, 128)], gsem)

                return carry2

            lax.fori_loop(0, GS, chunk_body, 0)

            for t in range(DEPTH):
                kk = GS - DEPTH + t
                rr = (kk % DEPTH) * 128
                pltpu.make_async_copy(rows_v.at[pl.ds(rr, 128)],
                                      acc_sh.at[dst_v.at[kk]], ssem).wait()
            return carry

        lax.fori_loop(0, NG, group_body, 0)
        plsc.subcore_barrier()

        pltpu.sync_copy(acc_sh.at[rows], out_hbm.at[c, rows])

    return segsum


def _pre_body(x_ref, w_ref, o_ref):
    o_ref[...] = _dot(x_ref[...], w_ref[...])


def _pre(x_pad, w):
    return pl.pallas_call(
        _pre_body,
        grid=(NBLK,),
        in_specs=[
            pl.BlockSpec((BLK, x_pad.shape[1]), lambda i: (i, 0)),
            pl.BlockSpec(w.shape, lambda i: (0, 0)),
        ],
        out_specs=pl.BlockSpec((BLK, w.shape[1]), lambda i: (i, 0)),
        out_shape=jax.ShapeDtypeStruct((NPAD, w.shape[1]), jnp.float32),
    )(x_pad, w)


def _mid1_body(a0_ref, a1_ref, x_ref, wr_ref, b_ref, o_ref):
    v = a0_ref[...] + a1_ref[...] + _dot(x_ref[...], wr_ref[...]) + b_ref[...]
    o_ref[...] = _elu(v)


def _mid1(a0, a1, x_pad, wroot, b):
    F = wroot.shape[1]
    return pl.pallas_call(
        _mid1_body,
        grid=(NBLK,),
        in_specs=[
            pl.BlockSpec((BLK, F), lambda i: (i, 0)),
            pl.BlockSpec((BLK, F), lambda i: (i, 0)),
            pl.BlockSpec((BLK, x_pad.shape[1]), lambda i: (i, 0)),
            pl.BlockSpec(wroot.shape, lambda i: (0, 0)),
            pl.BlockSpec((1, F), lambda i: (0, 0)),
        ],
        out_specs=pl.BlockSpec((BLK, F), lambda i: (i, 0)),
        out_shape=jax.ShapeDtypeStruct((NPAD, F), jnp.float32),
    )(a0, a1, x_pad, wroot, b.reshape(1, F))


def _mid2_body(a0_ref, a1_ref, h_ref, wrel_ref, wroot_ref, b_ref,
               lo_ref, hi_ref):
    agg = a0_ref[...] + a1_ref[...]
    v = _dot(agg, wrel_ref[...]) + _dot(h_ref[...], wroot_ref[...]) + b_ref[...]
    h2 = _elu(v)
    lo_ref[...] = h2[:, :64]
    hi_ref[...] = h2[:, 64:]


def _mid2(a0, a1, h, wrel, wroot, b):
    return pl.pallas_call(
        _mid2_body,
        grid=(NBLK,),
        in_specs=[
            pl.BlockSpec((BLK, 64), lambda i: (i, 0)),
            pl.BlockSpec((BLK, 64), lambda i: (i, 0)),
            pl.BlockSpec((BLK, 64), lambda i: (i, 0)),
            pl.BlockSpec((64, 128), lambda i: (0, 0)),
            pl.BlockSpec((64, 128), lambda i: (0, 0)),
            pl.BlockSpec((1, 128), lambda i: (0, 0)),
        ],
        out_specs=[
            pl.BlockSpec((BLK, 64), lambda i: (i, 0)),
            pl.BlockSpec((BLK, 64), lambda i: (i, 0)),
        ],
        out_shape=[
            jax.ShapeDtypeStruct((NPAD, 64), jnp.float32),
            jax.ShapeDtypeStruct((NPAD, 64), jnp.float32),
        ],
    )(a0, a1, h, wrel, wroot, b.reshape(1, 128))


def _pool_body(a0lo_ref, a1lo_ref, a0hi_ref, a1hi_ref, hlo_ref, hhi_ref,
               wrlo_ref, wrhi_ref, wolo_ref, wohi_ref, b_ref, bat_ref,
               sums_ref, cnts_ref):
    i = pl.program_id(0)
    v = (_dot(a0lo_ref[...] + a1lo_ref[...], wrlo_ref[...])
         + _dot(a0hi_ref[...] + a1hi_ref[...], wrhi_ref[...])
         + _dot(hlo_ref[...], wolo_ref[...])
         + _dot(hhi_ref[...], wohi_ref[...])
         + b_ref[...])
    h3 = _elu(v)
    bat = bat_ref[pl.ds(i * BLK, BLK)]
    onehot = (bat.reshape(BLK, 1) ==
              lax.broadcasted_iota(jnp.int32, (1, G), 1)).astype(jnp.float32)

    @pl.when(i == 0)
    def _():
        sums_ref[...] = jnp.zeros_like(sums_ref)
        cnts_ref[...] = jnp.zeros_like(cnts_ref)

    sums_ref[...] += _dot(onehot.T, h3)
    cnts_ref[...] += jnp.sum(onehot, axis=0).reshape(G, 1) + jnp.zeros(
        (G, 128), jnp.float32)


def _pool(a0lo, a1lo, a0hi, a1hi, hlo, hhi, wrel, wroot, b, batch_pad):
    half = pl.BlockSpec((BLK, 64), lambda i: (i, 0))
    wspec = pl.BlockSpec((64, 128), lambda i: (0, 0))
    return pl.pallas_call(
        _pool_body,
        grid=(NBLK,),
        in_specs=[half, half, half, half, half, half,
                  wspec, wspec, wspec, wspec,
                  pl.BlockSpec((1, 128), lambda i: (0, 0)),
                  pl.BlockSpec((NPAD,), lambda i: (0,))],
        out_specs=[
            pl.BlockSpec((G, 128), lambda i: (0, 0)),
            pl.BlockSpec((G, 128), lambda i: (0, 0)),
        ],
        out_shape=[
            jax.ShapeDtypeStruct((G, 128), jnp.float32),
            jax.ShapeDtypeStruct((G, 128), jnp.float32),
        ],
    )(a0lo, a1lo, a0hi, a1hi, hlo, hhi,
      wrel[:64], wrel[64:], wroot[:64], wroot[64:],
      b.reshape(1, 128), batch_pad)


def _head_body(sums_ref, cnts_ref, w1_ref, b1_ref, w2_ref, b2_ref,
               w3_ref, b3_ref, o_ref):
    cnt = jnp.maximum(cnts_ref[:, 0:1], 1.0)
    m = sums_ref[...] / cnt
    z = _elu(_dot(m, w1_ref[...]) + b1_ref[...])
    z = _elu(_dot(z, w2_ref[...]) + b2_ref[...])
    z = _dot(z, w3_ref[...]) + b3_ref[...]
    zmax = jnp.max(z, axis=1, keepdims=True)
    lse = zmax + jnp.log(jnp.sum(jnp.exp(z - zmax), axis=1, keepdims=True))
    o_ref[...] = z - lse


def _head(sums, cnts, w1, b1, w2, b2, w3, b3):
    return pl.pallas_call(
        _head_body,
        out_shape=jax.ShapeDtypeStruct((G, C), jnp.float32),
    )(sums, cnts, w1, b1.reshape(1, -1), w2, b2.reshape(1, -1),
      w3, b3.reshape(1, -1))


def kernel(x, edge_index, batch,
           conv1_Wrel, conv1_Wroot, conv1_b,
           conv2_Wrel, conv2_Wroot, conv2_b,
           conv3_Wrel, conv3_Wroot, conv3_b,
           mlp1_W, mlp1_b, mlp2_W, mlp2_b, mlp3_W, mlp3_b):
    x_pad = jnp.pad(x, ((0, NPAD - N), (0, 0)))
    batch_pad = jnp.pad(batch, (0, NPAD - N), constant_values=G)

    src = jnp.pad(edge_index[0], (0, EPAD - E)).reshape(32 * CH, 128)
    dst = jnp.pad(edge_index[1], (0, EPAD - E),
                  constant_values=N).reshape(32 * CH, 128)
    zeros64 = jnp.zeros((ROWS_PER_TILE, 64), jnp.float32)

    seg = _make_segsum()

    y1 = _pre(x_pad, conv1_Wrel)
    p1 = seg(y1, src, dst, zeros64)
    h1 = _mid1(p1[0], p1[1], x_pad, conv1_Wroot, conv1_b)

    p2 = seg(h1, src, dst, zeros64)
    h2lo, h2hi = _mid2(p2[0], p2[1], h1, conv2_Wrel, conv2_Wroot, conv2_b)

    p3lo = seg(h2lo, src, dst, zeros64)
    p3hi = seg(h2hi, src, dst, zeros64)
    sums, cnts = _pool(p3lo[0], p3lo[1], p3hi[0], p3hi[1], h2lo, h2hi,
                       conv3_Wrel, conv3_Wroot, conv3_b, batch_pad)

    return _head(sums, cnts, mlp1_W, mlp1_b, mlp2_W, mlp2_b, mlp3_W, mlp3_b)


# speedup vs baseline: 9.2695x; 2.0663x over previous
import functools

import jax
import jax.numpy as jnp
from jax import lax
from jax.experimental import pallas as pl
from jax.experimental.pallas import tpu as pltpu
from jax.experimental.pallas import tpu_sc as plsc

N = 10000
E = 320000
D = 128
H = 64
C = 10
G = 64

NPAD = 10240
ROWS_PER_TILE = NPAD // 16
CH = 80
EPAD = 32 * CH * 128
GS = 16
NG = CH // GS
DEPTH = 2
BLK = 1024
NBLK = NPAD // BLK

_HIGH = jax.lax.Precision.HIGHEST


def _dot(a, b):
    return jnp.dot(a, b, precision=_HIGH, preferred_element_type=jnp.float32)


def _elu(v):
    return jnp.where(v > 0, v, jnp.exp(jnp.minimum(v, 0.0)) - 1.0)


@functools.lru_cache(maxsize=None)
def _make_segsum():
    F = 64
    mesh = plsc.VectorSubcoreMesh(core_axis_name="c", subcore_axis_name="s")

    @functools.partial(
        pl.kernel,
        out_type=jax.ShapeDtypeStruct((2, NPAD, F), jnp.float32),
        mesh=mesh,
        scratch_types=[
            pltpu.VMEM((GS, 128), jnp.int32),
            pltpu.VMEM((GS, 128), jnp.int32),
            pltpu.VMEM((DEPTH * 128, F), jnp.float32),
            pltpu.VMEM_SHARED((NPAD, F), jnp.float32),
            pltpu.VMEM_SHARED((NPAD, F), jnp.float32),
            pltpu.SemaphoreType.DMA,
            pltpu.SemaphoreType.DMA,
        ],
        compiler_params=pltpu.CompilerParams(use_tc_tiling_on_sc=False),
    )
    def segsum(table_hbm, srcm_hbm, dstm_hbm, zeros_hbm, out_hbm,
               src_v, dst_v, rows_v, tab_sh, acc_sh, gsem, ssem):
        c = lax.axis_index("c")
        s = lax.axis_index("s")
        tile = c * 16 + s
        rows = pl.ds(s * ROWS_PER_TILE, ROWS_PER_TILE)

        pltpu.sync_copy(table_hbm.at[rows], tab_sh.at[rows])
        pltpu.sync_copy(zeros_hbm, acc_sh.at[rows])
        plsc.subcore_barrier()

        def group_body(g, carry):
            base = tile * CH + g * GS
            pltpu.sync_copy(srcm_hbm.at[pl.ds(base, GS)], src_v)
            pltpu.sync_copy(dstm_hbm.at[pl.ds(base, GS)], dst_v)
            for k in range(DEPTH - 1):
                pltpu.async_copy(tab_sh.at[src_v.at[k]],
                                 rows_v.at[pl.ds(k * 128, 128)], gsem)

            def chunk_body(k, carry2):
                r0 = (k % DEPTH) * 128
                pltpu.make_async_copy(tab_sh.at[src_v.at[k]],
                                      rows_v.at[pl.ds(r0, 128)], gsem).wait()
                pltpu.async_copy(rows_v.at[pl.ds(r0, 128)],
                                 acc_sh.at[dst_v.at[k]], ssem, add=True)
                kn = k + DEPTH - 1

                @pl.when((kn < GS) & (k >= 1))
                def _():
                    rp = ((k - 1) % DEPTH) * 128
                    pltpu.make_async_copy(rows_v.at[pl.ds(rp, 128)],
                                          acc_sh.at[dst_v.at[k - 1]],
                                          ssem).wait()

                @pl.when(kn < GS)
                def _():
                    rn = (kn % DEPTH) * 128
                    pltpu.async_copy(tab_sh.at[src_v.at[kn]],
                                     rows_v.at[pl.ds(rn, 128)], gsem)

                return carry2

            lax.fori_loop(0, GS, chunk_body, 0)

            for t in range(DEPTH):
                kk = GS - DEPTH + t
                rr = (kk % DEPTH) * 128
                pltpu.make_async_copy(rows_v.at[pl.ds(rr, 128)],
                                      acc_sh.at[dst_v.at[kk]], ssem).wait()
            return carry

        lax.fori_loop(0, NG, group_body, 0)
        plsc.subcore_barrier()

        pltpu.sync_copy(acc_sh.at[rows], out_hbm.at[c, rows])

    return segsum


def _pre_body(x_ref, w_ref, o_ref):
    o_ref[...] = _dot(x_ref[...], w_ref[...])


def _pre(x_pad, w):
    return pl.pallas_call(
        _pre_body,
        grid=(NBLK,),
        in_specs=[
            pl.BlockSpec((BLK, x_pad.shape[1]), lambda i: (i, 0)),
            pl.BlockSpec(w.shape, lambda i: (0, 0)),
        ],
        out_specs=pl.BlockSpec((BLK, w.shape[1]), lambda i: (i, 0)),
        out_shape=jax.ShapeDtypeStruct((NPAD, w.shape[1]), jnp.float32),
    )(x_pad, w)


def _mid1_body(a0_ref, a1_ref, x_ref, wr_ref, b_ref, o_ref):
    v = a0_ref[...] + a1_ref[...] + _dot(x_ref[...], wr_ref[...]) + b_ref[...]
    o_ref[...] = _elu(v)


def _mid1(a0, a1, x_pad, wroot, b):
    F = wroot.shape[1]
    return pl.pallas_call(
        _mid1_body,
        grid=(NBLK,),
        in_specs=[
            pl.BlockSpec((BLK, F), lambda i: (i, 0)),
            pl.BlockSpec((BLK, F), lambda i: (i, 0)),
            pl.BlockSpec((BLK, x_pad.shape[1]), lambda i: (i, 0)),
            pl.BlockSpec(wroot.shape, lambda i: (0, 0)),
            pl.BlockSpec((1, F), lambda i: (0, 0)),
        ],
        out_specs=pl.BlockSpec((BLK, F), lambda i: (i, 0)),
        out_shape=jax.ShapeDtypeStruct((NPAD, F), jnp.float32),
    )(a0, a1, x_pad, wroot, b.reshape(1, F))


def _mid2_body(a0_ref, a1_ref, h_ref, wrel_ref, wroot_ref, b_ref,
               lo_ref, hi_ref):
    agg = a0_ref[...] + a1_ref[...]
    v = _dot(agg, wrel_ref[...]) + _dot(h_ref[...], wroot_ref[...]) + b_ref[...]
    h2 = _elu(v)
    lo_ref[...] = h2[:, :64]
    hi_ref[...] = h2[:, 64:]


def _mid2(a0, a1, h, wrel, wroot, b):
    return pl.pallas_call(
        _mid2_body,
        grid=(NBLK,),
        in_specs=[
            pl.BlockSpec((BLK, 64), lambda i: (i, 0)),
            pl.BlockSpec((BLK, 64), lambda i: (i, 0)),
            pl.BlockSpec((BLK, 64), lambda i: (i, 0)),
            pl.BlockSpec((64, 128), lambda i: (0, 0)),
            pl.BlockSpec((64, 128), lambda i: (0, 0)),
            pl.BlockSpec((1, 128), lambda i: (0, 0)),
        ],
        out_specs=[
            pl.BlockSpec((BLK, 64), lambda i: (i, 0)),
            pl.BlockSpec((BLK, 64), lambda i: (i, 0)),
        ],
        out_shape=[
            jax.ShapeDtypeStruct((NPAD, 64), jnp.float32),
            jax.ShapeDtypeStruct((NPAD, 64), jnp.float32),
        ],
    )(a0, a1, h, wrel, wroot, b.reshape(1, 128))


def _pool_body(a0lo_ref, a1lo_ref, a0hi_ref, a1hi_ref, hlo_ref, hhi_ref,
               wrlo_ref, wrhi_ref, wolo_ref, wohi_ref, b_ref, bat_ref,
               sums_ref, cnts_ref):
    i = pl.program_id(0)
    v = (_dot(a0lo_ref[...] + a1lo_ref[...], wrlo_ref[...])
         + _dot(a0hi_ref[...] + a1hi_ref[...], wrhi_ref[...])
         + _dot(hlo_ref[...], wolo_ref[...])
         + _dot(hhi_ref[...], wohi_ref[...])
         + b_ref[...])
    h3 = _elu(v)
    bat = bat_ref[pl.ds(i * BLK, BLK)]
    onehot = (bat.reshape(BLK, 1) ==
              lax.broadcasted_iota(jnp.int32, (1, G), 1)).astype(jnp.float32)

    @pl.when(i == 0)
    def _():
        sums_ref[...] = jnp.zeros_like(sums_ref)
        cnts_ref[...] = jnp.zeros_like(cnts_ref)

    sums_ref[...] += _dot(onehot.T, h3)
    cnts_ref[...] += jnp.sum(onehot, axis=0).reshape(G, 1) + jnp.zeros(
        (G, 128), jnp.float32)


def _pool(a0lo, a1lo, a0hi, a1hi, hlo, hhi, wrel, wroot, b, batch_pad):
    half = pl.BlockSpec((BLK, 64), lambda i: (i, 0))
    wspec = pl.BlockSpec((64, 128), lambda i: (0, 0))
    return pl.pallas_call(
        _pool_body,
        grid=(NBLK,),
        in_specs=[half, half, half, half, half, half,
                  wspec, wspec, wspec, wspec,
                  pl.BlockSpec((1, 128), lambda i: (0, 0)),
                  pl.BlockSpec((NPAD,), lambda i: (0,))],
        out_specs=[
            pl.BlockSpec((G, 128), lambda i: (0, 0)),
            pl.BlockSpec((G, 128), lambda i: (0, 0)),
        ],
        out_shape=[
            jax.ShapeDtypeStruct((G, 128), jnp.float32),
            jax.ShapeDtypeStruct((G, 128), jnp.float32),
        ],
    )(a0lo, a1lo, a0hi, a1hi, hlo, hhi,
      wrel[:64], wrel[64:], wroot[:64], wroot[64:],
      b.reshape(1, 128), batch_pad)


def _head_body(sums_ref, cnts_ref, w1_ref, b1_ref, w2_ref, b2_ref,
               w3_ref, b3_ref, o_ref):
    cnt = jnp.maximum(cnts_ref[:, 0:1], 1.0)
    m = sums_ref[...] / cnt
    z = _elu(_dot(m, w1_ref[...]) + b1_ref[...])
    z = _elu(_dot(z, w2_ref[...]) + b2_ref[...])
    z = _dot(z, w3_ref[...]) + b3_ref[...]
    zmax = jnp.max(z, axis=1, keepdims=True)
    lse = zmax + jnp.log(jnp.sum(jnp.exp(z - zmax), axis=1, keepdims=True))
    o_ref[...] = z - lse


def _head(sums, cnts, w1, b1, w2, b2, w3, b3):
    return pl.pallas_call(
        _head_body,
        out_shape=jax.ShapeDtypeStruct((G, C), jnp.float32),
    )(sums, cnts, w1, b1.reshape(1, -1), w2, b2.reshape(1, -1),
      w3, b3.reshape(1, -1))


def kernel(x, edge_index, batch,
           conv1_Wrel, conv1_Wroot, conv1_b,
           conv2_Wrel, conv2_Wroot, conv2_b,
           conv3_Wrel, conv3_Wroot, conv3_b,
           mlp1_W, mlp1_b, mlp2_W, mlp2_b, mlp3_W, mlp3_b):
    x_pad = jnp.pad(x, ((0, NPAD - N), (0, 0)))
    batch_pad = jnp.pad(batch, (0, NPAD - N), constant_values=G)

    src = jnp.pad(edge_index[0], (0, EPAD - E)).reshape(32 * CH, 128)
    dst = jnp.pad(edge_index[1], (0, EPAD - E),
                  constant_values=N).reshape(32 * CH, 128)
    zeros64 = jnp.zeros((ROWS_PER_TILE, 64), jnp.float32)

    seg = _make_segsum()

    y1 = _pre(x_pad, conv1_Wrel)
    p1 = seg(y1, src, dst, zeros64)
    h1 = _mid1(p1[0], p1[1], x_pad, conv1_Wroot, conv1_b)

    p2 = seg(h1, src, dst, zeros64)
    h2lo, h2hi = _mid2(p2[0], p2[1], h1, conv2_Wrel, conv2_Wroot, conv2_b)

    p3lo = seg(h2lo, src, dst, zeros64)
    p3hi = seg(h2hi, src, dst, zeros64)
    sums, cnts = _pool(p3lo[0], p3lo[1], p3hi[0], p3hi[1], h2lo, h2hi,
                       conv3_Wrel, conv3_Wroot, conv3_b, batch_pad)

    return _head(sums, cnts, mlp1_W, mlp1_b, mlp2_W, mlp2_b, mlp3_W, mlp3_b)


# speedup vs baseline: 9.6597x; 1.0421x over previous
import functools

import jax
import jax.numpy as jnp
from jax import lax
from jax.experimental import pallas as pl
from jax.experimental.pallas import tpu as pltpu
from jax.experimental.pallas import tpu_sc as plsc

N = 10000
E = 320000
D = 128
H = 64
C = 10
G = 64

NPAD = 10240
ROWS_PER_TILE = NPAD // 16
CH = 80
EPAD = 32 * CH * 128
GS = 16
NG = CH // GS
DEPTH = 2
BLK = 1024
NBLK = NPAD // BLK

_HIGH = jax.lax.Precision.DEFAULT


def _dot(a, b):
    return jnp.dot(a, b, precision=_HIGH, preferred_element_type=jnp.float32)


def _elu(v):
    return jnp.where(v > 0, v, jnp.exp(jnp.minimum(v, 0.0)) - 1.0)


@functools.lru_cache(maxsize=None)
def _make_segsum():
    F = 64
    mesh = plsc.VectorSubcoreMesh(core_axis_name="c", subcore_axis_name="s")

    @functools.partial(
        pl.kernel,
        out_type=jax.ShapeDtypeStruct((2, NPAD, F), jnp.float32),
        mesh=mesh,
        scratch_types=[
            pltpu.VMEM((GS, 128), jnp.int32),
            pltpu.VMEM((GS, 128), jnp.int32),
            pltpu.VMEM((DEPTH * 128, F), jnp.float32),
            pltpu.VMEM_SHARED((NPAD, F), jnp.float32),
            pltpu.VMEM_SHARED((NPAD, F), jnp.float32),
            pltpu.SemaphoreType.DMA,
            pltpu.SemaphoreType.DMA,
        ],
        compiler_params=pltpu.CompilerParams(use_tc_tiling_on_sc=False),
    )
    def segsum(table_hbm, srcm_hbm, dstm_hbm, zeros_hbm, out_hbm,
               src_v, dst_v, rows_v, tab_sh, acc_sh, gsem, ssem):
        c = lax.axis_index("c")
        s = lax.axis_index("s")
        tile = c * 16 + s
        rows = pl.ds(s * ROWS_PER_TILE, ROWS_PER_TILE)

        pltpu.sync_copy(table_hbm.at[rows], tab_sh.at[rows])
        pltpu.sync_copy(zeros_hbm, acc_sh.at[rows])
        plsc.subcore_barrier()

        def group_body(g, carry):
            base = tile * CH + g * GS
            pltpu.sync_copy(srcm_hbm.at[pl.ds(base, GS)], src_v)
            pltpu.sync_copy(dstm_hbm.at[pl.ds(base, GS)], dst_v)
            for k in range(DEPTH - 1):
                pltpu.async_copy(tab_sh.at[src_v.at[k]],
                                 rows_v.at[pl.ds(k * 128, 128)], gsem)

            def chunk_body(k, carry2):
                r0 = (k % DEPTH) * 128
                pltpu.make_async_copy(tab_sh.at[src_v.at[k]],
                                      rows_v.at[pl.ds(r0, 128)], gsem).wait()
                pltpu.async_copy(rows_v.at[pl.ds(r0, 128)],
                                 acc_sh.at[dst_v.at[k]], ssem, add=True)
                kn = k + DEPTH - 1

                @pl.when((kn < GS) & (k >= 1))
                def _():
                    rp = ((k - 1) % DEPTH) * 128
                    pltpu.make_async_copy(rows_v.at[pl.ds(rp, 128)],
                                          acc_sh.at[dst_v.at[k - 1]],
                                          ssem).wait()

                @pl.when(kn < GS)
                def _():
                    rn = (kn % DEPTH) * 128
                    pltpu.async_copy(tab_sh.at[src_v.at[kn]],
                                     rows_v.at[pl.ds(rn, 128)], gsem)

                return carry2

            lax.fori_loop(0, GS, chunk_body, 0)

            for t in range(DEPTH):
                kk = GS - DEPTH + t
                rr = (kk % DEPTH) * 128
                pltpu.make_async_copy(rows_v.at[pl.ds(rr, 128)],
                                      acc_sh.at[dst_v.at[kk]], ssem).wait()
            return carry

        lax.fori_loop(0, NG, group_body, 0)
        plsc.subcore_barrier()

        pltpu.sync_copy(acc_sh.at[rows], out_hbm.at[c, rows])

    return segsum


def _pre_body(x_ref, w_ref, o_ref):
    o_ref[...] = _dot(x_ref[...], w_ref[...])


def _pre(x_pad, w):
    return pl.pallas_call(
        _pre_body,
        grid=(NBLK,),
        in_specs=[
            pl.BlockSpec((BLK, x_pad.shape[1]), lambda i: (i, 0)),
            pl.BlockSpec(w.shape, lambda i: (0, 0)),
        ],
        out_specs=pl.BlockSpec((BLK, w.shape[1]), lambda i: (i, 0)),
        out_shape=jax.ShapeDtypeStruct((NPAD, w.shape[1]), jnp.float32),
    )(x_pad, w)


def _mid1_body(a0_ref, a1_ref, x_ref, wr_ref, b_ref, o_ref):
    v = a0_ref[...] + a1_ref[...] + _dot(x_ref[...], wr_ref[...]) + b_ref[...]
    o_ref[...] = _elu(v)


def _mid1(a0, a1, x_pad, wroot, b):
    F = wroot.shape[1]
    return pl.pallas_call(
        _mid1_body,
        grid=(NBLK,),
        in_specs=[
            pl.BlockSpec((BLK, F), lambda i: (i, 0)),
            pl.BlockSpec((BLK, F), lambda i: (i, 0)),
            pl.BlockSpec((BLK, x_pad.shape[1]), lambda i: (i, 0)),
            pl.BlockSpec(wroot.shape, lambda i: (0, 0)),
            pl.BlockSpec((1, F), lambda i: (0, 0)),
        ],
        out_specs=pl.BlockSpec((BLK, F), lambda i: (i, 0)),
        out_shape=jax.ShapeDtypeStruct((NPAD, F), jnp.float32),
    )(a0, a1, x_pad, wroot, b.reshape(1, F))


def _mid2_body(a0_ref, a1_ref, h_ref, wrel_ref, wroot_ref, b_ref,
               lo_ref, hi_ref):
    agg = a0_ref[...] + a1_ref[...]
    v = _dot(agg, wrel_ref[...]) + _dot(h_ref[...], wroot_ref[...]) + b_ref[...]
    h2 = _elu(v)
    lo_ref[...] = h2[:, :64]
    hi_ref[...] = h2[:, 64:]


def _mid2(a0, a1, h, wrel, wroot, b):
    return pl.pallas_call(
        _mid2_body,
        grid=(NBLK,),
        in_specs=[
            pl.BlockSpec((BLK, 64), lambda i: (i, 0)),
            pl.BlockSpec((BLK, 64), lambda i: (i, 0)),
            pl.BlockSpec((BLK, 64), lambda i: (i, 0)),
            pl.BlockSpec((64, 128), lambda i: (0, 0)),
            pl.BlockSpec((64, 128), lambda i: (0, 0)),
            pl.BlockSpec((1, 128), lambda i: (0, 0)),
        ],
        out_specs=[
            pl.BlockSpec((BLK, 64), lambda i: (i, 0)),
            pl.BlockSpec((BLK, 64), lambda i: (i, 0)),
        ],
        out_shape=[
            jax.ShapeDtypeStruct((NPAD, 64), jnp.float32),
            jax.ShapeDtypeStruct((NPAD, 64), jnp.float32),
        ],
    )(a0, a1, h, wrel, wroot, b.reshape(1, 128))


def _pool_body(a0lo_ref, a1lo_ref, a0hi_ref, a1hi_ref, hlo_ref, hhi_ref,
               wrlo_ref, wrhi_ref, wolo_ref, wohi_ref, b_ref, bat_ref,
               sums_ref, cnts_ref):
    i = pl.program_id(0)
    v = (_dot(a0lo_ref[...] + a1lo_ref[...], wrlo_ref[...])
         + _dot(a0hi_ref[...] + a1hi_ref[...], wrhi_ref[...])
         + _dot(hlo_ref[...], wolo_ref[...])
         + _dot(hhi_ref[...], wohi_ref[...])
         + b_ref[...])
    h3 = _elu(v)
    bat = bat_ref[pl.ds(i * BLK, BLK)]
    onehot = (bat.reshape(BLK, 1) ==
              lax.broadcasted_iota(jnp.int32, (1, G), 1)).astype(jnp.float32)

    @pl.when(i == 0)
    def _():
        sums_ref[...] = jnp.zeros_like(sums_ref)
        cnts_ref[...] = jnp.zeros_like(cnts_ref)

    sums_ref[...] += _dot(onehot.T, h3)
    cnts_ref[...] += jnp.sum(onehot, axis=0).reshape(G, 1) + jnp.zeros(
        (G, 128), jnp.float32)


def _pool(a0lo, a1lo, a0hi, a1hi, hlo, hhi, wrel, wroot, b, batch_pad):
    half = pl.BlockSpec((BLK, 64), lambda i: (i, 0))
    wspec = pl.BlockSpec((64, 128), lambda i: (0, 0))
    return pl.pallas_call(
        _pool_body,
        grid=(NBLK,),
        in_specs=[half, half, half, half, half, half,
                  wspec, wspec, wspec, wspec,
                  pl.BlockSpec((1, 128), lambda i: (0, 0)),
                  pl.BlockSpec((NPAD,), lambda i: (0,))],
        out_specs=[
            pl.BlockSpec((G, 128), lambda i: (0, 0)),
            pl.BlockSpec((G, 128), lambda i: (0, 0)),
        ],
        out_shape=[
            jax.ShapeDtypeStruct((G, 128), jnp.float32),
            jax.ShapeDtypeStruct((G, 128), jnp.float32),
        ],
    )(a0lo, a1lo, a0hi, a1hi, hlo, hhi,
      wrel[:64], wrel[64:], wroot[:64], wroot[64:],
      b.reshape(1, 128), batch_pad)


def _head_body(sums_ref, cnts_ref, w1_ref, b1_ref, w2_ref, b2_ref,
               w3_ref, b3_ref, o_ref):
    cnt = jnp.maximum(cnts_ref[:, 0:1], 1.0)
    m = sums_ref[...] / cnt
    z = _elu(_dot(m, w1_ref[...]) + b1_ref[...])
    z = _elu(_dot(z, w2_ref[...]) + b2_ref[...])
    z = _dot(z, w3_ref[...]) + b3_ref[...]
    zmax = jnp.max(z, axis=1, keepdims=True)
    lse = zmax + jnp.log(jnp.sum(jnp.exp(z - zmax), axis=1, keepdims=True))
    o_ref[...] = z - lse


def _head(sums, cnts, w1, b1, w2, b2, w3, b3):
    return pl.pallas_call(
        _head_body,
        out_shape=jax.ShapeDtypeStruct((G, C), jnp.float32),
    )(sums, cnts, w1, b1.reshape(1, -1), w2, b2.reshape(1, -1),
      w3, b3.reshape(1, -1))


def kernel(x, edge_index, batch,
           conv1_Wrel, conv1_Wroot, conv1_b,
           conv2_Wrel, conv2_Wroot, conv2_b,
           conv3_Wrel, conv3_Wroot, conv3_b,
           mlp1_W, mlp1_b, mlp2_W, mlp2_b, mlp3_W, mlp3_b):
    x_pad = jnp.pad(x, ((0, NPAD - N), (0, 0)))
    batch_pad = jnp.pad(batch, (0, NPAD - N), constant_values=G)

    src = jnp.pad(edge_index[0], (0, EPAD - E)).reshape(32 * CH, 128)
    dst = jnp.pad(edge_index[1], (0, EPAD - E),
                  constant_values=N).reshape(32 * CH, 128)
    zeros64 = jnp.zeros((ROWS_PER_TILE, 64), jnp.float32)

    seg = _make_segsum()

    y1 = _pre(x_pad, conv1_Wrel)
    p1 = seg(y1, src, dst, zeros64)
    h1 = _mid1(p1[0], p1[1], x_pad, conv1_Wroot, conv1_b)

    p2 = seg(h1, src, dst, zeros64)
    h2lo, h2hi = _mid2(p2[0], p2[1], h1, conv2_Wrel, conv2_Wroot, conv2_b)

    p3lo = seg(h2lo, src, dst, zeros64)
    p3hi = seg(h2hi, src, dst, zeros64)
    sums, cnts = _pool(p3lo[0], p3lo[1], p3hi[0], p3hi[1], h2lo, h2hi,
                       conv3_Wrel, conv3_Wroot, conv3_b, batch_pad)

    return _head(sums, cnts, mlp1_W, mlp1_b, mlp2_W, mlp2_b, mlp3_W, mlp3_b)


# speedup vs baseline: 10.3389x; 1.0703x over previous
import functools

import jax
import jax.numpy as jnp
from jax import lax
from jax.experimental import pallas as pl
from jax.experimental.pallas import tpu as pltpu
from jax.experimental.pallas import tpu_sc as plsc

N = 10000
E = 320000
D = 128
H = 64
C = 10
G = 64

NPAD = 10240
ROWS_PER_TILE = NPAD // 16
CH = 80
EPAD = 32 * CH * 128
GS = 16
NG = CH // GS
DEPTH = 2
BLK = 1024
NBLK = NPAD // BLK

_HIGH = jax.lax.Precision.DEFAULT


def _dot(a, b):
    return jnp.dot(a, b, precision=_HIGH, preferred_element_type=jnp.float32)


def _elu(v):
    return jnp.where(v > 0, v, jnp.exp(jnp.minimum(v, 0.0)) - 1.0)


@functools.lru_cache(maxsize=None)
def _make_segsum(ntab):
    F = 64
    mesh = plsc.VectorSubcoreMesh(core_axis_name="c", subcore_axis_name="s")

    @functools.partial(
        pl.kernel,
        out_type=[jax.ShapeDtypeStruct((2, NPAD, F), jnp.float32)] * ntab,
        mesh=mesh,
        scratch_types=[
            pltpu.VMEM((GS, 128), jnp.int32),
            pltpu.VMEM((GS, 128), jnp.int32),
            pltpu.VMEM((DEPTH * 128, F), jnp.float32),
            pltpu.VMEM_SHARED((NPAD, F), jnp.float32),
            pltpu.VMEM_SHARED((NPAD, F), jnp.float32),
            pltpu.SemaphoreType.DMA,
            pltpu.SemaphoreType.DMA,
        ],
        compiler_params=pltpu.CompilerParams(use_tc_tiling_on_sc=False),
    )
    def segsum(*refs):
        tabs_hbm = refs[:ntab]
        srcm_hbm, dstm_hbm, zeros_hbm = refs[ntab:ntab + 3]
        outs_hbm = refs[ntab + 3:2 * ntab + 3]
        src_v, dst_v, rows_v, tab_sh, acc_sh, gsem, ssem = refs[2 * ntab + 3:]
        c = lax.axis_index("c")
        s = lax.axis_index("s")
        tile = c * 16 + s
        rows = pl.ds(s * ROWS_PER_TILE, ROWS_PER_TILE)

        def group_body(g, carry):
            base = tile * CH + g * GS
            pltpu.sync_copy(srcm_hbm.at[pl.ds(base, GS)], src_v)
            pltpu.sync_copy(dstm_hbm.at[pl.ds(base, GS)], dst_v)
            for k in range(DEPTH - 1):
                pltpu.async_copy(tab_sh.at[src_v.at[k]],
                                 rows_v.at[pl.ds(k * 128, 128)], gsem)

            def chunk_body(k, carry2):
                r0 = (k % DEPTH) * 128
                pltpu.make_async_copy(tab_sh.at[src_v.at[k]],
                                      rows_v.at[pl.ds(r0, 128)], gsem).wait()
                pltpu.async_copy(rows_v.at[pl.ds(r0, 128)],
                                 acc_sh.at[dst_v.at[k]], ssem, add=True)
                kn = k + DEPTH - 1

                @pl.when((kn < GS) & (k >= 1))
                def _():
                    rp = ((k - 1) % DEPTH) * 128
                    pltpu.make_async_copy(rows_v.at[pl.ds(rp, 128)],
                                          acc_sh.at[dst_v.at[k - 1]],
                                          ssem).wait()

                @pl.when(kn < GS)
                def _():
                    rn = (kn % DEPTH) * 128
                    pltpu.async_copy(tab_sh.at[src_v.at[kn]],
                                     rows_v.at[pl.ds(rn, 128)], gsem)

                return carry2

            lax.fori_loop(0, GS, chunk_body, 0)

            for t in range(DEPTH):
                kk = GS - DEPTH + t
                rr = (kk % DEPTH) * 128
                pltpu.make_async_copy(rows_v.at[pl.ds(rr, 128)],
                                      acc_sh.at[dst_v.at[kk]], ssem).wait()
            return carry

        for tab_hbm, out_hbm in zip(tabs_hbm, outs_hbm):
            pltpu.sync_copy(tab_hbm.at[rows], tab_sh.at[rows])
            pltpu.sync_copy(zeros_hbm, acc_sh.at[rows])
            plsc.subcore_barrier()
            lax.fori_loop(0, NG, group_body, 0)
            plsc.subcore_barrier()
            pltpu.sync_copy(acc_sh.at[rows], out_hbm.at[c, rows])

    return segsum


def _pre_body(x_ref, wrel_ref, wroot_ref, b_ref, y_ref, r_ref):
    x = x_ref[...]
    y_ref[...] = _dot(x, wrel_ref[...])
    r_ref[...] = _dot(x, wroot_ref[...]) + b_ref[...]


def _pre(x_pad, wrel, wroot, b):
    F = wrel.shape[1]
    return pl.pallas_call(
        _pre_body,
        grid=(NBLK,),
        in_specs=[
            pl.BlockSpec((BLK, x_pad.shape[1]), lambda i: (i, 0)),
            pl.BlockSpec(wrel.shape, lambda i: (0, 0)),
            pl.BlockSpec(wroot.shape, lambda i: (0, 0)),
            pl.BlockSpec((1, F), lambda i: (0, 0)),
        ],
        out_specs=[
            pl.BlockSpec((BLK, F), lambda i: (i, 0)),
            pl.BlockSpec((BLK, F), lambda i: (i, 0)),
        ],
        out_shape=[
            jax.ShapeDtypeStruct((NPAD, F), jnp.float32),
            jax.ShapeDtypeStruct((NPAD, F), jnp.float32),
        ],
    )(x_pad, wrel, wroot, b.reshape(1, F))


def _mid1_body(p_ref, r_ref, o_ref):
    o_ref[...] = _elu(p_ref[0] + p_ref[1] + r_ref[...])


def _mid1(p, r):
    F = r.shape[1]
    return pl.pallas_call(
        _mid1_body,
        grid=(NBLK,),
        in_specs=[
            pl.BlockSpec((2, BLK, F), lambda i: (0, i, 0)),
            pl.BlockSpec((BLK, F), lambda i: (i, 0)),
        ],
        out_specs=pl.BlockSpec((BLK, F), lambda i: (i, 0)),
        out_shape=jax.ShapeDtypeStruct((NPAD, F), jnp.float32),
    )(p, r)


def _mid2_body(p_ref, h_ref, wrel_ref, wroot_ref, b_ref, lo_ref, hi_ref):
    agg = p_ref[0] + p_ref[1]
    v = _dot(agg, wrel_ref[...]) + _dot(h_ref[...], wroot_ref[...]) + b_ref[...]
    h2 = _elu(v)
    lo_ref[...] = h2[:, :64]
    hi_ref[...] = h2[:, 64:]


def _mid2(p, h, wrel, wroot, b):
    return pl.pallas_call(
        _mid2_body,
        grid=(NBLK,),
        in_specs=[
            pl.BlockSpec((2, BLK, 64), lambda i: (0, i, 0)),
            pl.BlockSpec((BLK, 64), lambda i: (i, 0)),
            pl.BlockSpec((64, 128), lambda i: (0, 0)),
            pl.BlockSpec((64, 128), lambda i: (0, 0)),
            pl.BlockSpec((1, 128), lambda i: (0, 0)),
        ],
        out_specs=[
            pl.BlockSpec((BLK, 64), lambda i: (i, 0)),
            pl.BlockSpec((BLK, 64), lambda i: (i, 0)),
        ],
        out_shape=[
            jax.ShapeDtypeStruct((NPAD, 64), jnp.float32),
            jax.ShapeDtypeStruct((NPAD, 64), jnp.float32),
        ],
    )(p, h, wrel, wroot, b.reshape(1, 128))


def _pool_body(plo_ref, phi_ref, hlo_ref, hhi_ref,
               wrlo_ref, wrhi_ref, wolo_ref, wohi_ref, b_ref, bat_ref,
               w1_ref, b1_ref, w2_ref, b2_ref, w3_ref, b3_ref,
               o_ref, sums_ref, cnts_ref):
    i = pl.program_id(0)
    v = (_dot(plo_ref[0] + plo_ref[1], wrlo_ref[...])
         + _dot(phi_ref[0] + phi_ref[1], wrhi_ref[...])
         + _dot(hlo_ref[...], wolo_ref[...])
         + _dot(hhi_ref[...], wohi_ref[...])
         + b_ref[...])
    h3 = _elu(v)
    bat = bat_ref[pl.ds(i * BLK, BLK)]
    onehot = (bat.reshape(BLK, 1) ==
              lax.broadcasted_iota(jnp.int32, (1, G), 1)).astype(jnp.float32)

    @pl.when(i == 0)
    def _():
        sums_ref[...] = jnp.zeros_like(sums_ref)
        cnts_ref[...] = jnp.zeros_like(cnts_ref)

    sums_ref[...] += _dot(onehot.T, h3)
    cnts_ref[...] += jnp.sum(onehot, axis=0).reshape(G, 1) + jnp.zeros(
        (G, 128), jnp.float32)

    @pl.when(i == NBLK - 1)
    def _():
        cnt = jnp.maximum(cnts_ref[:, 0:1], 1.0)
        m = sums_ref[...] / cnt
        z = _elu(_dot(m, w1_ref[...]) + b1_ref[...])
        z = _elu(_dot(z, w2_ref[...]) + b2_ref[...])
        z = _dot(z, w3_ref[...]) + b3_ref[...]
        zmax = jnp.max(z, axis=1, keepdims=True)
        lse = zmax + jnp.log(jnp.sum(jnp.exp(z - zmax), axis=1, keepdims=True))
        o_ref[...] = z - lse


def _pool_head(plo, phi, hlo, hhi, wrel, wroot, b, batch_pad,
               w1, b1, w2, b2, w3, b3):
    half3 = pl.BlockSpec((2, BLK, 64), lambda i: (0, i, 0))
    half = pl.BlockSpec((BLK, 64), lambda i: (i, 0))
    wspec = pl.BlockSpec((64, 128), lambda i: (0, 0))
    const = lambda shape: pl.BlockSpec(shape, lambda i: tuple(0 for _ in shape))
    return pl.pallas_call(
        _pool_body,
        grid=(NBLK,),
        in_specs=[half3, half3, half, half,
                  wspec, wspec, wspec, wspec,
                  const((1, 128)),
                  pl.BlockSpec((NPAD,), lambda i: (0,)),
                  const(w1.shape), const((1, 128)),
                  const(w2.shape), const((1, 64)),
                  const(w3.shape), const((1, C))],
        out_specs=pl.BlockSpec((G, C), lambda i: (0, 0)),
        out_shape=jax.ShapeDtypeStruct((G, C), jnp.float32),
        scratch_shapes=[
            pltpu.VMEM((G, 128), jnp.float32),
            pltpu.VMEM((G, 128), jnp.float32),
        ],
    )(plo, phi, hlo, hhi,
      wrel[:64], wrel[64:], wroot[:64], wroot[64:],
      b.reshape(1, 128), batch_pad,
      w1, b1.reshape(1, -1), w2, b2.reshape(1, -1), w3, b3.reshape(1, -1))


def kernel(x, edge_index, batch,
           conv1_Wrel, conv1_Wroot, conv1_b,
           conv2_Wrel, conv2_Wroot, conv2_b,
           conv3_Wrel, conv3_Wroot, conv3_b,
           mlp1_W, mlp1_b, mlp2_W, mlp2_b, mlp3_W, mlp3_b):
    x_pad = jnp.pad(x, ((0, NPAD - N), (0, 0)))
    batch_pad = jnp.pad(batch, (0, NPAD - N), constant_values=G)

    src = jnp.pad(edge_index[0], (0, EPAD - E)).reshape(32 * CH, 128)
    dst = jnp.pad(edge_index[1], (0, EPAD - E),
                  constant_values=N).reshape(32 * CH, 128)
    zeros64 = jnp.zeros((ROWS_PER_TILE, 64), jnp.float32)

    seg1 = _make_segsum(1)
    seg2 = _make_segsum(2)

    y1, r1 = _pre(x_pad, conv1_Wrel, conv1_Wroot, conv1_b)
    (p1,) = seg1(y1, src, dst, zeros64)
    h1 = _mid1(p1, r1)

    (p2,) = seg1(h1, src, dst, zeros64)
    h2lo, h2hi = _mid2(p2, h1, conv2_Wrel, conv2_Wroot, conv2_b)

    p3lo, p3hi = seg2(h2lo, h2hi, src, dst, zeros64)
    return _pool_head(p3lo, p3hi, h2lo, h2hi,
                      conv3_Wrel, conv3_Wroot, conv3_b, batch_pad,
                      mlp1_W, mlp1_b, mlp2_W, mlp2_b, mlp3_W, mlp3_b)


# speedup vs baseline: 10.7962x; 1.0442x over previous
import functools

import jax
import jax.numpy as jnp
from jax import lax
from jax.experimental import pallas as pl
from jax.experimental.pallas import tpu as pltpu
from jax.experimental.pallas import tpu_sc as plsc

N = 10000
E = 320000
D = 128
H = 64
C = 10
G = 64

NPAD = 10240
ROWS_PER_TILE = NPAD // 16
CH = 80
EPAD = 32 * CH * 128
GS = 40
NG = CH // GS
DEPTH = 2
BLK = 1024
NBLK = NPAD // BLK

_HIGH = jax.lax.Precision.DEFAULT


def _dot(a, b):
    return jnp.dot(a, b, precision=_HIGH, preferred_element_type=jnp.float32)


def _elu(v):
    return jnp.where(v > 0, v, jnp.exp(jnp.minimum(v, 0.0)) - 1.0)


@functools.lru_cache(maxsize=None)
def _make_segsum(ntab):
    F = 64
    mesh = plsc.VectorSubcoreMesh(core_axis_name="c", subcore_axis_name="s")

    @functools.partial(
        pl.kernel,
        out_type=[jax.ShapeDtypeStruct((2, NPAD, F), jnp.float32)] * ntab,
        mesh=mesh,
        scratch_types=[
            pltpu.VMEM((GS, 128), jnp.int32),
            pltpu.VMEM((GS, 128), jnp.int32),
            pltpu.VMEM((DEPTH * 128, F), jnp.float32),
            pltpu.VMEM_SHARED((NPAD, F), jnp.float32),
            pltpu.VMEM_SHARED((NPAD, F), jnp.float32),
            pltpu.SemaphoreType.DMA,
            pltpu.SemaphoreType.DMA,
        ],
        compiler_params=pltpu.CompilerParams(use_tc_tiling_on_sc=False),
    )
    def segsum(*refs):
        tabs_hbm = refs[:ntab]
        srcm_hbm, dstm_hbm, zeros_hbm = refs[ntab:ntab + 3]
        outs_hbm = refs[ntab + 3:2 * ntab + 3]
        src_v, dst_v, rows_v, tab_sh, acc_sh, gsem, ssem = refs[2 * ntab + 3:]
        c = lax.axis_index("c")
        s = lax.axis_index("s")
        tile = c * 16 + s
        rows = pl.ds(s * ROWS_PER_TILE, ROWS_PER_TILE)

        def group_body(g, carry):
            base = tile * CH + g * GS
            pltpu.sync_copy(srcm_hbm.at[pl.ds(base, GS)], src_v)
            pltpu.sync_copy(dstm_hbm.at[pl.ds(base, GS)], dst_v)
            for k in range(DEPTH - 1):
                pltpu.async_copy(tab_sh.at[src_v.at[k]],
                                 rows_v.at[pl.ds(k * 128, 128)], gsem)

            def chunk_body(k, carry2):
                r0 = (k % DEPTH) * 128
                pltpu.make_async_copy(tab_sh.at[src_v.at[k]],
                                      rows_v.at[pl.ds(r0, 128)], gsem).wait()
                pltpu.async_copy(rows_v.at[pl.ds(r0, 128)],
                                 acc_sh.at[dst_v.at[k]], ssem, add=True)
                kn = k + DEPTH - 1

                @pl.when((kn < GS) & (k >= 1))
                def _():
                    rp = ((k - 1) % DEPTH) * 128
                    pltpu.make_async_copy(rows_v.at[pl.ds(rp, 128)],
                                          acc_sh.at[dst_v.at[k - 1]],
                                          ssem).wait()

                @pl.when(kn < GS)
                def _():
                    rn = (kn % DEPTH) * 128
                    pltpu.async_copy(tab_sh.at[src_v.at[kn]],
                                     rows_v.at[pl.ds(rn, 128)], gsem)

                return carry2

            lax.fori_loop(0, GS, chunk_body, 0)

            for t in range(DEPTH):
                kk = GS - DEPTH + t
                rr = (kk % DEPTH) * 128
                pltpu.make_async_copy(rows_v.at[pl.ds(rr, 128)],
                                      acc_sh.at[dst_v.at[kk]], ssem).wait()
            return carry

        for tab_hbm, out_hbm in zip(tabs_hbm, outs_hbm):
            pltpu.sync_copy(tab_hbm.at[rows], tab_sh.at[rows])
            pltpu.sync_copy(zeros_hbm, acc_sh.at[rows])
            plsc.subcore_barrier()
            lax.fori_loop(0, NG, group_body, 0)
            plsc.subcore_barrier()
            pltpu.sync_copy(acc_sh.at[rows], out_hbm.at[c, rows])

    return segsum


def _pre_body(x_ref, wrel_ref, wroot_ref, b_ref, y_ref, r_ref):
    x = x_ref[...]
    y_ref[...] = _dot(x, wrel_ref[...])
    r_ref[...] = _dot(x, wroot_ref[...]) + b_ref[...]


def _pre(x_pad, wrel, wroot, b):
    F = wrel.shape[1]
    return pl.pallas_call(
        _pre_body,
        grid=(NBLK,),
        in_specs=[
            pl.BlockSpec((BLK, x_pad.shape[1]), lambda i: (i, 0)),
            pl.BlockSpec(wrel.shape, lambda i: (0, 0)),
            pl.BlockSpec(wroot.shape, lambda i: (0, 0)),
            pl.BlockSpec((1, F), lambda i: (0, 0)),
        ],
        out_specs=[
            pl.BlockSpec((BLK, F), lambda i: (i, 0)),
            pl.BlockSpec((BLK, F), lambda i: (i, 0)),
        ],
        out_shape=[
            jax.ShapeDtypeStruct((NPAD, F), jnp.float32),
            jax.ShapeDtypeStruct((NPAD, F), jnp.float32),
        ],
    )(x_pad, wrel, wroot, b.reshape(1, F))


def _mid1_body(p_ref, r_ref, o_ref):
    o_ref[...] = _elu(p_ref[0] + p_ref[1] + r_ref[...])


def _mid1(p, r):
    F = r.shape[1]
    return pl.pallas_call(
        _mid1_body,
        grid=(NBLK,),
        in_specs=[
            pl.BlockSpec((2, BLK, F), lambda i: (0, i, 0)),
            pl.BlockSpec((BLK, F), lambda i: (i, 0)),
        ],
        out_specs=pl.BlockSpec((BLK, F), lambda i: (i, 0)),
        out_shape=jax.ShapeDtypeStruct((NPAD, F), jnp.float32),
    )(p, r)


def _mid2_body(p_ref, h_ref, wrel_ref, wroot_ref, b_ref, lo_ref, hi_ref):
    agg = p_ref[0] + p_ref[1]
    v = _dot(agg, wrel_ref[...]) + _dot(h_ref[...], wroot_ref[...]) + b_ref[...]
    h2 = _elu(v)
    lo_ref[...] = h2[:, :64]
    hi_ref[...] = h2[:, 64:]


def _mid2(p, h, wrel, wroot, b):
    return pl.pallas_call(
        _mid2_body,
        grid=(NBLK,),
        in_specs=[
            pl.BlockSpec((2, BLK, 64), lambda i: (0, i, 0)),
            pl.BlockSpec((BLK, 64), lambda i: (i, 0)),
            pl.BlockSpec((64, 128), lambda i: (0, 0)),
            pl.BlockSpec((64, 128), lambda i: (0, 0)),
            pl.BlockSpec((1, 128), lambda i: (0, 0)),
        ],
        out_specs=[
            pl.BlockSpec((BLK, 64), lambda i: (i, 0)),
            pl.BlockSpec((BLK, 64), lambda i: (i, 0)),
        ],
        out_shape=[
            jax.ShapeDtypeStruct((NPAD, 64), jnp.float32),
            jax.ShapeDtypeStruct((NPAD, 64), jnp.float32),
        ],
    )(p, h, wrel, wroot, b.reshape(1, 128))


def _pool_body(plo_ref, phi_ref, hlo_ref, hhi_ref,
               wrlo_ref, wrhi_ref, wolo_ref, wohi_ref, b_ref, bat_ref,
               w1_ref, b1_ref, w2_ref, b2_ref, w3_ref, b3_ref,
               o_ref, sums_ref, cnts_ref):
    i = pl.program_id(0)
    v = (_dot(plo_ref[0] + plo_ref[1], wrlo_ref[...])
         + _dot(phi_ref[0] + phi_ref[1], wrhi_ref[...])
         + _dot(hlo_ref[...], wolo_ref[...])
         + _dot(hhi_ref[...], wohi_ref[...])
         + b_ref[...])
    h3 = _elu(v)
    bat = bat_ref[pl.ds(i * BLK, BLK)]
    onehot = (bat.reshape(BLK, 1) ==
              lax.broadcasted_iota(jnp.int32, (1, G), 1)).astype(jnp.float32)

    @pl.when(i == 0)
    def _():
        sums_ref[...] = jnp.zeros_like(sums_ref)
        cnts_ref[...] = jnp.zeros_like(cnts_ref)

    sums_ref[...] += _dot(onehot.T, h3)
    cnts_ref[...] += jnp.sum(onehot, axis=0).reshape(G, 1) + jnp.zeros(
        (G, 128), jnp.float32)

    @pl.when(i == NBLK - 1)
    def _():
        cnt = jnp.maximum(cnts_ref[:, 0:1], 1.0)
        m = sums_ref[...] / cnt
        z = _elu(_dot(m, w1_ref[...]) + b1_ref[...])
        z = _elu(_dot(z, w2_ref[...]) + b2_ref[...])
        z = _dot(z, w3_ref[...]) + b3_ref[...]
        zmax = jnp.max(z, axis=1, keepdims=True)
        lse = zmax + jnp.log(jnp.sum(jnp.exp(z - zmax), axis=1, keepdims=True))
        o_ref[...] = z - lse


def _pool_head(plo, phi, hlo, hhi, wrel, wroot, b, batch_pad,
               w1, b1, w2, b2, w3, b3):
    half3 = pl.BlockSpec((2, BLK, 64), lambda i: (0, i, 0))
    half = pl.BlockSpec((BLK, 64), lambda i: (i, 0))
    wspec = pl.BlockSpec((64, 128), lambda i: (0, 0))
    const = lambda shape: pl.BlockSpec(shape, lambda i: tuple(0 for _ in shape))
    return pl.pallas_call(
        _pool_body,
        grid=(NBLK,),
        in_specs=[half3, half3, half, half,
                  wspec, wspec, wspec, wspec,
                  const((1, 128)),
                  pl.BlockSpec((NPAD,), lambda i: (0,)),
                  const(w1.shape), const((1, 128)),
                  const(w2.shape), const((1, 64)),
                  const(w3.shape), const((1, C))],
        out_specs=pl.BlockSpec((G, C), lambda i: (0, 0)),
        out_shape=jax.ShapeDtypeStruct((G, C), jnp.float32),
        scratch_shapes=[
            pltpu.VMEM((G, 128), jnp.float32),
            pltpu.VMEM((G, 128), jnp.float32),
        ],
    )(plo, phi, hlo, hhi,
      wrel[:64], wrel[64:], wroot[:64], wroot[64:],
      b.reshape(1, 128), batch_pad,
      w1, b1.reshape(1, -1), w2, b2.reshape(1, -1), w3, b3.reshape(1, -1))


def kernel(x, edge_index, batch,
           conv1_Wrel, conv1_Wroot, conv1_b,
           conv2_Wrel, conv2_Wroot, conv2_b,
           conv3_Wrel, conv3_Wroot, conv3_b,
           mlp1_W, mlp1_b, mlp2_W, mlp2_b, mlp3_W, mlp3_b):
    x_pad = jnp.pad(x, ((0, NPAD - N), (0, 0)))
    batch_pad = jnp.pad(batch, (0, NPAD - N), constant_values=G)

    src = jnp.pad(edge_index[0], (0, EPAD - E)).reshape(32 * CH, 128)
    dst = jnp.pad(edge_index[1], (0, EPAD - E),
                  constant_values=N).reshape(32 * CH, 128)
    zeros64 = jnp.zeros((ROWS_PER_TILE, 64), jnp.float32)

    seg1 = _make_segsum(1)
    seg2 = _make_segsum(2)

    y1, r1 = _pre(x_pad, conv1_Wrel, conv1_Wroot, conv1_b)
    (p1,) = seg1(y1, src, dst, zeros64)
    h1 = _mid1(p1, r1)

    (p2,) = seg1(h1, src, dst, zeros64)
    h2lo, h2hi = _mid2(p2, h1, conv2_Wrel, conv2_Wroot, conv2_b)

    p3lo, p3hi = seg2(h2lo, h2hi, src, dst, zeros64)
    return _pool_head(p3lo, p3hi, h2lo, h2hi,
                      conv3_Wrel, conv3_Wroot, conv3_b, batch_pad,
                      mlp1_W, mlp1_b, mlp2_W, mlp2_b, mlp3_W, mlp3_b)
